# baseline (speedup 1.0000x reference)
import functools

import jax
import jax.numpy as jnp
from jax import lax
from jax.experimental import pallas as pl
from jax.experimental.pallas import tpu as pltpu
from jax.experimental.pallas import tpu_sc as plsc

K = 150
KP = 160
NW = 32
WPB = 4
CH = 128


def _make_sc_kernel(npix_total, ppw):
    nchunks = ppw // CH
    mesh = plsc.VectorSubcoreMesh(core_axis_name="c", subcore_axis_name="s")

    @functools.partial(
        pl.kernel,
        mesh=mesh,
        compiler_params=pltpu.CompilerParams(use_tc_tiling_on_sc=False,
                                             needs_layout_passes=False),
        out_type=jax.ShapeDtypeStruct((npix_total * K,), jnp.float32),
        scratch_types=[
            pltpu.VMEM((K, KP), jnp.float32),
            pltpu.VMEM((K, KP), jnp.float32),
            pltpu.VMEM((ppw,), jnp.int32),
            pltpu.VMEM((ppw,), jnp.int32),
            pltpu.VMEM((CH * K + 16,), jnp.float32),
        ],
    )
    def sc_kernel(a_hbm, b_hbm, xt_hbm, x0_hbm, out_hbm, a_v, b_v, xt_v, x0_v, obuf):
        cid = lax.axis_index("c")
        sid = lax.axis_index("s")
        wid = sid * 2 + cid
        batch = wid // WPB
        row0 = wid * ppw
        pltpu.sync_copy(a_hbm.at[batch], a_v)
        pltpu.sync_copy(b_hbm.at[batch], b_v)
        pltpu.sync_copy(xt_hbm.at[pl.ds(row0, ppw)], xt_v)
        pltpu.sync_copy(x0_hbm.at[pl.ds(row0, ppw)], x0_v)

        lanes = lax.iota(jnp.int32, 16)
        offs = [lanes + j * 16 for j in range(K // 16 + 1)]
        tail_mask = lanes < (K % 16)

        def chunk_body(ci, carry):
            pbase = ci * CH

            def group_body(g, carry2):
                gbase = g * 16
                xt_vec = xt_v[pl.ds(pbase + gbase, 16)]
                x0_vec = x0_v[pl.ds(pbase + gbase, 16)]
                for lane in range(16):
                    xt = xt_vec[lane]
                    x0 = x0_vec[lane]
                    base = (gbase + lane) * K
                    prods = []
                    acc = None
                    for j in range(KP // 16):
                        av = a_v[xt, pl.ds(j * 16, 16)]
                        bv = b_v[x0, pl.ds(j * 16, 16)]
                        pv = av * bv
                        prods.append(pv)
                        acc = pv if acc is None else acc + pv
                    ssum = jnp.broadcast_to(jnp.sum(acc), (16,))
                    inv = 1.0 / (ssum + 1e-10)
                    for j in range(K // 16):
                        plsc.store_scatter(obuf, [base + offs[j]], prods[j] * inv)
                    plsc.store_scatter(obuf, [base + offs[K // 16]],
                                       prods[K // 16] * inv, mask=tail_mask)
                return carry2

            lax.fori_loop(0, CH // 16, group_body, 0)
            pltpu.sync_copy(obuf.at[pl.ds(0, CH * K)],
                            out_hbm.at[pl.ds((row0 + pbase) * K, CH * K)])
            return carry

        lax.fori_loop(0, nchunks, chunk_body, 0)

    return sc_kernel


def kernel(x_0, x_t, t, Q_t, Q_bar):
    Bc, Hc, Wc = x_0.shape
    Kc = Q_t.shape[-1]
    npix = Bc * Hc * Wc
    ppw = npix // NW
    tt = t.astype(jnp.int32)
    Qt_sel = Q_t[tt]
    tm1 = jnp.clip(tt - 1, 0, None)
    Qb_sel = Q_bar[tm1]
    eye = jnp.eye(Kc, dtype=jnp.float32)
    is0 = (tt == 0)[:, None, None]
    Qb_sel = jnp.where(is0, eye[None], Qb_sel)
    a_tab = jnp.pad(jnp.swapaxes(Qt_sel, 1, 2), ((0, 0), (0, 0), (0, KP - Kc)))
    b_tab = jnp.pad(jnp.swapaxes(Qb_sel, 1, 2), ((0, 0), (0, 0), (0, KP - Kc)))
    xt_flat = x_t.reshape(npix).astype(jnp.int32)
    x0_flat = x_0.reshape(npix).astype(jnp.int32)
    out = _make_sc_kernel(npix, ppw)(a_tab, b_tab, xt_flat, x0_flat)
    return out.reshape(Bc, Hc, Wc, Kc)


# speedup vs baseline: 1.3898x; 1.3898x over previous
import functools

import jax
import jax.numpy as jnp
from jax import lax
from jax.experimental import pallas as pl
from jax.experimental.pallas import tpu as pltpu
from jax.experimental.pallas import tpu_sc as plsc

K = 150
KP = 160
NW = 32
WPB = 4
CH = 128
TAIL = K - 16


def _make_sc_kernel(npix_total, ppw):
    nchunks = ppw // CH
    mesh = plsc.VectorSubcoreMesh(core_axis_name="c", subcore_axis_name="s")

    @functools.partial(
        pl.kernel,
        mesh=mesh,
        compiler_params=pltpu.CompilerParams(needs_layout_passes=False),
        out_type=jax.ShapeDtypeStruct((npix_total, K), jnp.float32),
        scratch_types=[
            pltpu.VMEM((K, KP), jnp.float32),
            pltpu.VMEM((K, KP), jnp.float32),
            pltpu.VMEM((ppw,), jnp.int32),
            pltpu.VMEM((ppw,), jnp.int32),
            pltpu.VMEM((CH, K), jnp.float32),
        ],
    )
    def sc_kernel(a_hbm, b_hbm, xt_hbm, x0_hbm, out_hbm, a_v, b_v, xt_v, x0_v, obuf):
        cid = lax.axis_index("c")
        sid = lax.axis_index("s")
        wid = sid * 2 + cid
        batch = wid // WPB
        row0 = wid * ppw
        pltpu.sync_copy(a_hbm.at[batch], a_v)
        pltpu.sync_copy(b_hbm.at[batch], b_v)
        pltpu.sync_copy(xt_hbm.at[pl.ds(row0, ppw)], xt_v)
        pltpu.sync_copy(x0_hbm.at[pl.ds(row0, ppw)], x0_v)

        def chunk_body(ci, carry):
            pbase = ci * CH

            def group_body(g, carry2):
                gbase = g * 16
                xt_vec = xt_v[pl.ds(pbase + gbase, 16)]
                x0_vec = x0_v[pl.ds(pbase + gbase, 16)]
                for lane in range(16):
                    xt = xt_vec[lane]
                    x0 = x0_vec[lane]
                    row = gbase + lane
                    prods = []
                    acc = None
                    for j in range(KP // 16):
                        av = a_v[xt, pl.ds(j * 16, 16)]
                        bv = b_v[x0, pl.ds(j * 16, 16)]
                        pv = av * bv
                        prods.append(pv)
                        acc = pv if acc is None else acc + pv
                    ssum = jnp.broadcast_to(jnp.sum(acc), (16,))
                    inv = 1.0 / (ssum + 1e-10)
                    for j in range(K // 16):
                        obuf[row, pl.ds(j * 16, 16)] = prods[j] * inv
                    tv = a_v[xt, pl.ds(TAIL, 16)] * b_v[x0, pl.ds(TAIL, 16)]
                    obuf[row, pl.ds(TAIL, 16)] = tv * inv
                return carry2

            lax.fori_loop(0, CH // 16, group_body, 0)
            pltpu.sync_copy(obuf, out_hbm.at[pl.ds(row0 + pbase, CH)])
            return carry

        lax.fori_loop(0, nchunks, chunk_body, 0)

    return sc_kernel


def kernel(x_0, x_t, t, Q_t, Q_bar):
    Bc, Hc, Wc = x_0.shape
    Kc = Q_t.shape[-1]
    npix = Bc * Hc * Wc
    ppw = npix // NW
    tt = t.astype(jnp.int32)
    Qt_sel = Q_t[tt]
    tm1 = jnp.clip(tt - 1, 0, None)
    Qb_sel = Q_bar[tm1]
    eye = jnp.eye(Kc, dtype=jnp.float32)
    is0 = (tt == 0)[:, None, None]
    Qb_sel = jnp.where(is0, eye[None], Qb_sel)
    a_tab = jnp.pad(jnp.swapaxes(Qt_sel, 1, 2), ((0, 0), (0, 0), (0, KP - Kc)))
    b_tab = jnp.pad(jnp.swapaxes(Qb_sel, 1, 2), ((0, 0), (0, 0), (0, KP - Kc)))
    xt_flat = x_t.reshape(npix).astype(jnp.int32)
    x0_flat = x_0.reshape(npix).astype(jnp.int32)
    out = _make_sc_kernel(npix, ppw)(a_tab, b_tab, xt_flat, x0_flat)
    return out.reshape(Bc, Hc, Wc, Kc)
